# baseline (speedup 1.0000x reference)
import functools

import jax
import jax.numpy as jnp
from jax import lax
from jax.experimental import pallas as pl
from jax.experimental.pallas import tpu as pltpu
from jax.experimental.pallas import tpu_sc as plsc

NUM_CORES = 2
NUM_SUBCORES = 16
NW = NUM_CORES * NUM_SUBCORES
LANES = 16
CHUNK = 128


def kernel(pos_h, pos_r, pos_t, entity_embds, rel_embds):
    B = pos_h.shape[0]
    E, D = entity_embds.shape
    R = rel_embds.shape[0]
    b_per_w = B // NW
    n_chunk = b_per_w // CHUNK
    rpb = 128 // D

    ent2 = entity_embds.reshape(E // rpb, 128)
    rel2 = rel_embds.reshape(R // rpb, 128)
    idx_h = pos_h.astype(jnp.int32)
    idx_r = pos_r.astype(jnp.int32)
    idx_t = pos_t.astype(jnp.int32)

    mesh = plsc.VectorSubcoreMesh(
        core_axis_name="c", subcore_axis_name="s",
        num_cores=NUM_CORES, num_subcores=NUM_SUBCORES,
    )

    out = jax.ShapeDtypeStruct((B, D), jnp.float32)

    @functools.partial(
        pl.kernel,
        out_type=(out, out, out),
        mesh=mesh,
        compiler_params=pltpu.CompilerParams(
            use_tc_tiling_on_sc=True, needs_layout_passes=False),
        scratch_types=[
            pltpu.VMEM((b_per_w,), jnp.int32),
            pltpu.VMEM((b_per_w,), jnp.int32),
            pltpu.VMEM((b_per_w,), jnp.int32),
            pltpu.VMEM((b_per_w,), jnp.int32),
            pltpu.VMEM((b_per_w,), jnp.int32),
            pltpu.VMEM((CHUNK, 128), jnp.float32),
            pltpu.VMEM((CHUNK, 128), jnp.float32),
            pltpu.VMEM((R // rpb, 128), jnp.float32),
            pltpu.VMEM((CHUNK, D), jnp.float32),
            pltpu.VMEM((CHUNK, D), jnp.float32),
            pltpu.VMEM((CHUNK, D), jnp.float32),
            pltpu.SemaphoreType.DMA,
            pltpu.SemaphoreType.DMA,
            pltpu.SemaphoreType.DMA,
        ],
    )
    def run(ih_hbm, ir_hbm, it_hbm, ent_hbm, rel_hbm,
            oh_hbm, or_hbm, ot_hbm,
            ih_v, ir_v, it_v, bh_v, bt_v, stg_h, stg_t, rel_v,
            ob_h, ob_r, ob_t, sem_h, sem_t, sem_out):
        wid = lax.axis_index("s") * NUM_CORES + lax.axis_index("c")
        base = wid * b_per_w

        pltpu.sync_copy(ih_hbm.at[pl.ds(base, b_per_w)], ih_v)
        pltpu.sync_copy(ir_hbm.at[pl.ds(base, b_per_w)], ir_v)
        pltpu.sync_copy(it_hbm.at[pl.ds(base, b_per_w)], it_v)
        pltpu.sync_copy(rel_hbm, rel_v)

        def mk_blk(idx_v, blk_v):
            def body(g, _):
                blk_v[pl.ds(g * LANES, LANES)] = (
                    idx_v[pl.ds(g * LANES, LANES)] >> 2)
                return 0
            lax.fori_loop(0, b_per_w // LANES, body, 0)

        mk_blk(ih_v, bh_v)
        mk_blk(it_v, bt_v)

        jmax = D // LANES

        def extract(idx_v, stg_v, ob_v, c):
            for g in range(CHUNK // LANES):
                rows = lax.iota(jnp.int32, LANES) + g * LANES
                off = (idx_v[pl.ds(c * CHUNK + g * LANES, LANES)] & (rpb - 1)) * D

                def body(j, _):
                    jj = jnp.full((LANES,), 0, jnp.int32) + j
                    vals = plsc.load_gather(stg_v, [rows, off + jj])
                    plsc.store_scatter(ob_v, [rows, jj], vals)
                    return 0

                lax.fori_loop(0, D, body, 0)

        def rel_extract(c):
            for g in range(CHUNK // LANES):
                rows = lax.iota(jnp.int32, LANES) + g * LANES
                rvec = ir_v[pl.ds(c * CHUNK + g * LANES, LANES)]
                blk = rvec >> 2
                off = (rvec & (rpb - 1)) * D

                def body(j, _):
                    jj = jnp.full((LANES,), 0, jnp.int32) + j
                    vals = plsc.load_gather(rel_v, [blk, off + jj])
                    plsc.store_scatter(ob_r, [rows, jj], vals)
                    return 0

                lax.fori_loop(0, D, body, 0)

        out_cps = []
        for c in range(n_chunk):
            cslice = pl.ds(c * CHUNK, CHUNK)
            cp_h = pltpu.async_copy(ent_hbm.at[bh_v.at[cslice]], stg_h, sem_h)
            cp_t = pltpu.async_copy(ent_hbm.at[bt_v.at[cslice]], stg_t, sem_t)
            rel_extract(c)
            dst = pl.ds(base + c * CHUNK, CHUNK)
            out_cps.append(pltpu.async_copy(ob_r, or_hbm.at[dst], sem_out))
            cp_h.wait()
            extract(ih_v, stg_h, ob_h, c)
            out_cps.append(pltpu.async_copy(ob_h, oh_hbm.at[dst], sem_out))
            cp_t.wait()
            extract(it_v, stg_t, ob_t, c)
            out_cps.append(pltpu.async_copy(ob_t, ot_hbm.at[dst], sem_out))
            for cp in out_cps:
                cp.wait()
            out_cps = []

    return run(idx_h, idx_r, idx_t, ent2, rel2)


# speedup vs baseline: 2.5224x; 2.5224x over previous
import functools

import jax
import jax.numpy as jnp
from jax import lax
from jax.experimental import pallas as pl
from jax.experimental.pallas import tpu as pltpu
from jax.experimental.pallas import tpu_sc as plsc

NUM_CORES = 2
NUM_SUBCORES = 16
NW = NUM_CORES * NUM_SUBCORES
LANES = 16
CHUNK = 256


def kernel(pos_h, pos_r, pos_t, entity_embds, rel_embds):
    B = pos_h.shape[0]
    E, D = entity_embds.shape
    R = rel_embds.shape[0]
    b_per_w = B // NW

    ent3 = entity_embds.reshape(E // 8, 8, D)
    idx_h = pos_h.astype(jnp.int32)
    idx_r = pos_r.astype(jnp.int32)
    idx_t = pos_t.astype(jnp.int32)

    mesh = plsc.VectorSubcoreMesh(
        core_axis_name="c", subcore_axis_name="s",
        num_cores=NUM_CORES, num_subcores=NUM_SUBCORES,
    )

    out = jax.ShapeDtypeStruct((B, D), jnp.float32)

    @functools.partial(
        pl.kernel,
        out_type=(out, out, out),
        mesh=mesh,
        compiler_params=pltpu.CompilerParams(
            use_tc_tiling_on_sc=True, needs_layout_passes=False),
        scratch_types=[
            pltpu.VMEM((b_per_w,), jnp.int32),
            pltpu.VMEM((b_per_w,), jnp.int32),
            pltpu.VMEM((b_per_w,), jnp.int32),
            pltpu.VMEM((CHUNK, 32), jnp.float32),
            pltpu.VMEM((CHUNK, 32), jnp.float32),
            pltpu.VMEM((CHUNK, 32), jnp.float32),
            pltpu.SemaphoreType.DMA,
            pltpu.SemaphoreType.DMA,
            pltpu.SemaphoreType.DMA,
            pltpu.SemaphoreType.DMA,
        ],
    )
    def run(ih_hbm, ir_hbm, it_hbm, ent_hbm, rel_hbm,
            oh_hbm, or_hbm, ot_hbm,
            ih_v, ir_v, it_v, rows_h, rows_r, rows_t,
            sem_h, sem_r, sem_t, sem_out):
        wid = lax.axis_index("s") * NUM_CORES + lax.axis_index("c")
        base = wid * b_per_w

        pltpu.sync_copy(ih_hbm.at[pl.ds(base, b_per_w)], ih_v)
        pltpu.sync_copy(ir_hbm.at[pl.ds(base, b_per_w)], ir_v)
        pltpu.sync_copy(it_hbm.at[pl.ds(base, b_per_w)], it_v)

        def fire_ent(idx_v, rows_v, sem, c):
            def body(g, _):
                vec = idx_v[pl.ds(c * CHUNK + g * LANES, LANES)]
                blk16 = vec >> 3
                sub16 = vec & 7
                for l in range(LANES):
                    lane = lax.iota(jnp.int32, LANES) == l
                    blk = lax.reduce_sum(jnp.where(lane, blk16, 0), axes=(0,))
                    sub = lax.reduce_sum(jnp.where(lane, sub16, 0), axes=(0,))
                    pltpu.async_copy(
                        ent_hbm.at[blk, sub],
                        rows_v.at[g * LANES + l], sem)
                return 0

            lax.fori_loop(0, CHUNK // LANES, body, 0)

        def fire_rel(idx_v, rows_v, sem, c):
            def body(g, _):
                vec = idx_v[pl.ds(c * CHUNK + g * LANES, LANES)]
                for l in range(LANES):
                    lane = lax.iota(jnp.int32, LANES) == l
                    row = lax.reduce_sum(jnp.where(lane, vec, 0), axes=(0,))
                    pltpu.async_copy(
                        rel_hbm.at[row], rows_v.at[g * LANES + l], sem)
                return 0

            lax.fori_loop(0, CHUNK // LANES, body, 0)

        def drain(rows_v, sem):
            pltpu.make_async_copy(
                oh_hbm.at[pl.ds(0, CHUNK)], rows_v, sem).wait()

        out_cps = []
        for c in range(b_per_w // CHUNK):
            fire_ent(ih_v, rows_h, sem_h, c)
            fire_ent(it_v, rows_t, sem_t, c)
            fire_rel(ir_v, rows_r, sem_r, c)
            dst = pl.ds(base + c * CHUNK, CHUNK)
            drain(rows_h, sem_h)
            out_cps.append(pltpu.async_copy(rows_h, oh_hbm.at[dst], sem_out))
            drain(rows_t, sem_t)
            out_cps.append(pltpu.async_copy(rows_t, ot_hbm.at[dst], sem_out))
            drain(rows_r, sem_r)
            out_cps.append(pltpu.async_copy(rows_r, or_hbm.at[dst], sem_out))
            for cp in out_cps:
                cp.wait()
            out_cps = []

    return run(idx_h, idx_r, idx_t, ent3, rel_embds)


# speedup vs baseline: 2.5857x; 1.0251x over previous
import functools

import jax
import jax.numpy as jnp
from jax import lax
from jax.experimental import pallas as pl
from jax.experimental.pallas import tpu as pltpu
from jax.experimental.pallas import tpu_sc as plsc

NUM_CORES = 2
NUM_SUBCORES = 16
NW = NUM_CORES * NUM_SUBCORES
LANES = 16
CHUNK = 256


def kernel(pos_h, pos_r, pos_t, entity_embds, rel_embds):
    B = pos_h.shape[0]
    E, D = entity_embds.shape
    R = rel_embds.shape[0]
    b_per_w = B // NW

    ent3 = entity_embds.reshape(E // 8, 8, D)
    idx_h = pos_h.astype(jnp.int32)
    idx_r = pos_r.astype(jnp.int32)
    idx_t = pos_t.astype(jnp.int32)

    mesh = plsc.VectorSubcoreMesh(
        core_axis_name="c", subcore_axis_name="s",
        num_cores=NUM_CORES, num_subcores=NUM_SUBCORES,
    )

    out = jax.ShapeDtypeStruct((D // 8, 8, B), jnp.float32)

    @functools.partial(
        pl.kernel,
        out_type=(out, out, out),
        mesh=mesh,
        compiler_params=pltpu.CompilerParams(
            use_tc_tiling_on_sc=True, needs_layout_passes=False),
        scratch_types=[
            pltpu.VMEM((b_per_w,), jnp.int32),
            pltpu.VMEM((b_per_w,), jnp.int32),
            pltpu.VMEM((b_per_w,), jnp.int32),
            pltpu.VMEM((CHUNK, 32), jnp.float32),
            pltpu.VMEM((CHUNK, 32), jnp.float32),
            pltpu.VMEM((CHUNK, 32), jnp.float32),
            pltpu.VMEM((4, 8, CHUNK), jnp.float32),
            pltpu.SemaphoreType.DMA,
            pltpu.SemaphoreType.DMA,
            pltpu.SemaphoreType.DMA,
            pltpu.SemaphoreType.DMA,
        ],
    )
    def run(ih_hbm, ir_hbm, it_hbm, ent_hbm, rel_hbm,
            oh_hbm, or_hbm, ot_hbm,
            ih_v, ir_v, it_v, rows_h, rows_r, rows_t, tbuf,
            sem_h, sem_r, sem_t, sem_out):
        wid = lax.axis_index("s") * NUM_CORES + lax.axis_index("c")
        base = wid * b_per_w

        pltpu.sync_copy(ih_hbm.at[pl.ds(base, b_per_w)], ih_v)
        pltpu.sync_copy(ir_hbm.at[pl.ds(base, b_per_w)], ir_v)
        pltpu.sync_copy(it_hbm.at[pl.ds(base, b_per_w)], it_v)

        def fire_ent(idx_v, rows_v, sem, c):
            def body(g, _):
                vec = idx_v[pl.ds(c * CHUNK + g * LANES, LANES)]
                blk16 = vec >> 3
                sub16 = vec & 7
                for l in range(LANES):
                    lane = lax.iota(jnp.int32, LANES) == l
                    blk = lax.reduce_sum(jnp.where(lane, blk16, 0), axes=(0,))
                    sub = lax.reduce_sum(jnp.where(lane, sub16, 0), axes=(0,))
                    pltpu.async_copy(
                        ent_hbm.at[blk, sub],
                        rows_v.at[g * LANES + l], sem)
                return 0

            lax.fori_loop(0, CHUNK // LANES, body, 0)

        def fire_rel(idx_v, rows_v, sem, c):
            def body(g, _):
                vec = idx_v[pl.ds(c * CHUNK + g * LANES, LANES)]
                for l in range(LANES):
                    lane = lax.iota(jnp.int32, LANES) == l
                    row = lax.reduce_sum(jnp.where(lane, vec, 0), axes=(0,))
                    pltpu.async_copy(
                        rel_hbm.at[row], rows_v.at[g * LANES + l], sem)
                return 0

            lax.fori_loop(0, CHUNK // LANES, body, 0)

        def drain(rows_v, sem):
            pltpu.make_async_copy(
                ent_hbm.at[pl.ds(0, CHUNK // 8)], rows_v, sem).wait()

        def flush(rows_v, out3, c):
            def jbody(j, _):
                jsplat = jnp.full((LANES,), 0, jnp.int32) + j
                gsplat = jsplat >> 3
                ssplat = jsplat & 7

                def wbody(w, _):
                    nvec = lax.iota(jnp.int32, LANES) + w * LANES
                    vals = plsc.load_gather(rows_v, [nvec, jsplat])
                    plsc.store_scatter(tbuf, [gsplat, ssplat, nvec], vals)
                    return 0

                lax.fori_loop(0, CHUNK // LANES, wbody, 0)
                return 0

            lax.fori_loop(0, D, jbody, 0)
            off = pl.multiple_of(base + c * CHUNK, 128)
            pltpu.sync_copy(tbuf, out3.at[:, :, pl.ds(off, CHUNK)])

        for c in range(b_per_w // CHUNK):
            fire_ent(ih_v, rows_h, sem_h, c)
            fire_ent(it_v, rows_t, sem_t, c)
            fire_rel(ir_v, rows_r, sem_r, c)
            drain(rows_h, sem_h)
            flush(rows_h, oh_hbm, c)
            drain(rows_t, sem_t)
            flush(rows_t, ot_hbm, c)
            drain(rows_r, sem_r)
            flush(rows_r, or_hbm, c)

    oh3, or3, ot3 = run(idx_h, idx_r, idx_t, ent3, rel_embds)
    return (oh3.reshape(D, B).T, or3.reshape(D, B).T, ot3.reshape(D, B).T)
